# baseline (speedup 1.0000x reference)
import functools

import jax
import jax.numpy as jnp
from jax import lax
from jax.experimental import pallas as pl
from jax.experimental.pallas import tpu as pltpu
from jax.experimental.pallas import tpu_sc as plsc

N = 10000
E = 160000
F = 256
H = 512
G = 64

NP = 10240
TILES = 16
ROWS_PT = NP // TILES
EPT = 10240
WIN = 128
NWIN = EPT // WIN
HWIN = NWIN // 2
BN = 256
GRID = NP // BN


def _make_prop(cpc, wc, out_w):
    mesh = plsc.VectorSubcoreMesh(core_axis_name="c", subcore_axis_name="s",
                                  num_cores=2, num_subcores=16)
    dense_out = wc % 128 == 0
    out_shape = (NP, out_w) if dense_out else (2 * cpc, NP, wc)

    @functools.partial(
        pl.kernel,
        out_type=jax.ShapeDtypeStruct(out_shape, jnp.float32),
        mesh=mesh,
        scratch_types=[
            pltpu.VMEM((HWIN, WIN), jnp.int32),
            pltpu.VMEM((HWIN, WIN), jnp.int32),
            pltpu.VMEM((2, WIN, wc), jnp.float32),
            pltpu.VMEM_SHARED((NP, wc), jnp.float32),
            pltpu.SemaphoreType.DMA,
            pltpu.SemaphoreType.DMA,
            pltpu.SemaphoreType.DMA,
            pltpu.SemaphoreType.DMA,
        ],
    )
    def prop(table, srcp, dstp, out, src_v, dst_v, gbuf, acc,
             sem0, sem1, sem2, sem3):
        c = lax.axis_index("c")
        s = lax.axis_index("s")
        r0 = s * ROWS_PT
        sems = (sem0, sem1)
        ssems = (sem2, sem3)
        for k in range(cpc):
            ch = c * cpc + k
            pltpu.sync_copy(table.at[ch, pl.ds(r0, ROWS_PT), :],
                            acc.at[pl.ds(r0, ROWS_PT), :])
            plsc.subcore_barrier()

            def start_g(j, b):
                pltpu.async_copy(table.at[ch].at[src_v.at[j]],
                                 gbuf.at[b], sems[b])

            def wait_g(b):
                pltpu.make_async_copy(table.at[ch].at[src_v.at[0]],
                                      gbuf.at[b], sems[b]).wait()

            def start_s(j, b):
                pltpu.async_copy(gbuf.at[b], acc.at[dst_v.at[j]],
                                 ssems[b], add=True)

            def wait_s(b):
                pltpu.make_async_copy(gbuf.at[b], acc.at[dst_v.at[0]],
                                      ssems[b]).wait()

            for h in range(2):
                pltpu.sync_copy(srcp.at[s, pl.ds(h * HWIN, HWIN)], src_v)
                pltpu.sync_copy(dstp.at[s, pl.ds(h * HWIN, HWIN)], dst_v)
                start_g(0, 0)
                start_g(1, 1)

                def win_body(w2, carry):
                    w = 2 * w2
                    for b in range(2):
                        wait_g(b)
                        start_s(w + b, b)
                    for b in range(2):
                        wait_s(b)
                        start_g(jnp.minimum(w + 2 + b, HWIN - 1), b)
                    return carry

                lax.fori_loop(0, HWIN // 2, win_body, 0)
                wait_g(0)
                wait_g(1)
            plsc.subcore_barrier()
            if dense_out:
                pltpu.sync_copy(acc.at[pl.ds(r0, ROWS_PT), :],
                                out.at[pl.ds(r0, ROWS_PT), pl.ds(ch * wc, wc)])
            else:
                pltpu.sync_copy(acc.at[pl.ds(r0, ROWS_PT), :],
                                out.at[ch, pl.ds(r0, ROWS_PT), :])
            plsc.subcore_barrier()

    return prop


def _make_deg():
    mesh = plsc.VectorSubcoreMesh(core_axis_name="c", subcore_axis_name="s",
                                  num_cores=2, num_subcores=16)

    @functools.partial(
        pl.kernel,
        out_type=jax.ShapeDtypeStruct((NP, 128), jnp.float32),
        mesh=mesh,
        scratch_types=[
            pltpu.VMEM((NWIN, WIN), jnp.int32),
            pltpu.VMEM((WIN, 128), jnp.float32),
            pltpu.VMEM_SHARED((NP, 128), jnp.float32),
            pltpu.SemaphoreType.DMA,
            pltpu.SemaphoreType.DMA,
        ],
    )
    def degk(ones_tab, dstp, out, dst_v, gbuf, acc, sem0, sem1):
        c = lax.axis_index("c")
        s = lax.axis_index("s")
        pltpu.sync_copy(dstp.at[s], dst_v)
        pltpu.sync_copy(ones_tab.at[pl.ds(0, WIN), :], gbuf)
        r0 = s * ROWS_PT
        pltpu.sync_copy(ones_tab.at[pl.ds(r0, ROWS_PT), :],
                        acc.at[pl.ds(r0, ROWS_PT), :])
        plsc.subcore_barrier()
        ssems = (sem0, sem1)

        def start_s(j, b):
            pltpu.async_copy(gbuf, acc.at[dst_v.at[j]], ssems[b], add=True)

        def wait_s(b):
            pltpu.make_async_copy(gbuf, acc.at[dst_v.at[0]], ssems[b]).wait()

        start_s(0, 0)
        start_s(1, 1)

        def win_body(w2, carry):
            w = 2 * w2
            for b in range(2):
                wait_s(b)
                start_s(w + 2 + b, b)
            return carry

        lax.fori_loop(0, NWIN // 2 - 1, win_body, 0)
        wait_s(0)
        wait_s(1)
        plsc.subcore_barrier()

        @pl.when(c == 0)
        def _():
            pltpu.sync_copy(acc.at[pl.ds(r0, ROWS_PT), :],
                            out.at[pl.ds(r0, ROWS_PT), :])

    return degk


_PROP_CACHE = {}


def _get_prop(cpc, wc, out_w):
    key = (cpc, wc, out_w)
    if key not in _PROP_CACHE:
        _PROP_CACHE[key] = _make_prop(cpc, wc, out_w) if cpc else _make_deg()
    return _PROP_CACHE[key]


def _scale_chunk_body(x_ref, deg_ref, o_ref):
    dinv = lax.rsqrt(deg_ref[:, 0:1])
    x = x_ref[...]
    o_ref[0] = x[:, :128] * dinv
    o_ref[1] = x[:, 128:] * dinv


def _scale_chunk(x_pad, deg):
    return pl.pallas_call(
        _scale_chunk_body,
        grid=(GRID,),
        in_specs=[pl.BlockSpec((BN, F), lambda i: (i, 0)),
                  pl.BlockSpec((BN, 16), lambda i: (i, 0))],
        out_specs=pl.BlockSpec((2, BN, 128), lambda i: (0, i, 0)),
        out_shape=jax.ShapeDtypeStruct((2, NP, 128), jnp.float32),
    )(x_pad, deg)


def _mm1_body(u_ref, w_ref, b_ref, deg_ref, o_ref):
    dinv = lax.rsqrt(deg_ref[:, 0:1])
    y = jnp.dot(u_ref[...] * dinv, w_ref[...],
                preferred_element_type=jnp.float32)
    o_ref[...] = y + b_ref[...]


def _mm1(u0, w1, b1, deg):
    return pl.pallas_call(
        _mm1_body,
        grid=(GRID,),
        in_specs=[pl.BlockSpec((BN, F), lambda i: (i, 0)),
                  pl.BlockSpec((F, H), lambda i: (0, 0)),
                  pl.BlockSpec((1, H), lambda i: (0, 0)),
                  pl.BlockSpec((BN, 16), lambda i: (i, 0))],
        out_specs=pl.BlockSpec((BN, H), lambda i: (i, 0)),
        out_shape=jax.ShapeDtypeStruct((NP, H), jnp.float32),
    )(u0, w1, b1, deg)


def _mm_chunk_body(h_ref, w_ref, deg_ref, o_ref):
    z = jnp.dot(h_ref[...], w_ref[...], preferred_element_type=jnp.float32)
    z = z * lax.rsqrt(deg_ref[:, 0:1])
    for j in range(4):
        o_ref[j] = z[:, j * 128:(j + 1) * 128]


def _mm_chunk(h, w, deg):
    return pl.pallas_call(
        _mm_chunk_body,
        grid=(GRID,),
        in_specs=[pl.BlockSpec((BN, H), lambda i: (i, 0)),
                  pl.BlockSpec((H, H), lambda i: (0, 0)),
                  pl.BlockSpec((BN, 16), lambda i: (i, 0))],
        out_specs=pl.BlockSpec((4, BN, 128), lambda i: (0, i, 0)),
        out_shape=jax.ShapeDtypeStruct((4, NP, 128), jnp.float32),
    )(h, w, deg)


def _stats_body(y_ref, deg_ref, b_ref, o_ref, *, scale):
    i = pl.program_id(0)
    y = y_ref[...]
    if scale:
        y = y * lax.rsqrt(deg_ref[:, 0:1]) + b_ref[...]
    rows = lax.broadcasted_iota(jnp.int32, (BN, 1), 0) + i * BN
    y = jnp.where(rows < N, y, 0.0)

    @pl.when(i == 0)
    def _():
        o_ref[...] = jnp.zeros((8, H), jnp.float32)

    s1 = jnp.sum(y, axis=0)[None, :]
    s2 = jnp.sum(y * y, axis=0)[None, :]
    o_ref[0:2, :] += jnp.concatenate([s1, s2], axis=0)


def _stats(y, deg, b, scale):
    return pl.pallas_call(
        functools.partial(_stats_body, scale=scale),
        grid=(GRID,),
        in_specs=[pl.BlockSpec((BN, H), lambda i: (i, 0)),
                  pl.BlockSpec((BN, 16), lambda i: (i, 0)),
                  pl.BlockSpec((1, H), lambda i: (0, 0))],
        out_specs=pl.BlockSpec((8, H), lambda i: (0, 0)),
        out_shape=jax.ShapeDtypeStruct((8, H), jnp.float32),
    )(y, deg, b)


def _apply_body(y_ref, deg_ref, b_ref, st_ref, g_ref, be_ref, o_ref,
                *, scale, relu):
    i = pl.program_id(0)
    y = y_ref[...]
    if scale:
        y = y * lax.rsqrt(deg_ref[:, 0:1]) + b_ref[...]
    mu = st_ref[0:1, :] / N
    var = st_ref[1:2, :] / N - mu * mu
    out = g_ref[...] * (y - mu) * lax.rsqrt(var + 1e-5) + be_ref[...]
    if relu:
        out = jnp.maximum(out, 0.0)
    rows = lax.broadcasted_iota(jnp.int32, (BN, 1), 0) + i * BN
    o_ref[...] = jnp.where(rows < N, out, 0.0)


def _apply(y, deg, b, st, g, be, scale, relu):
    return pl.pallas_call(
        functools.partial(_apply_body, scale=scale, relu=relu),
        grid=(GRID,),
        in_specs=[pl.BlockSpec((BN, H), lambda i: (i, 0)),
                  pl.BlockSpec((BN, 16), lambda i: (i, 0)),
                  pl.BlockSpec((1, H), lambda i: (0, 0)),
                  pl.BlockSpec((8, H), lambda i: (0, 0)),
                  pl.BlockSpec((1, H), lambda i: (0, 0)),
                  pl.BlockSpec((1, H), lambda i: (0, 0))],
        out_specs=pl.BlockSpec((BN, H), lambda i: (i, 0)),
        out_shape=jax.ShapeDtypeStruct((NP, H), jnp.float32),
    )(y, deg, b, st, g, be)


def _pool_body(h_ref, oh_ref, sum_ref, cnt_ref):
    i = pl.program_id(0)

    @pl.when(i == 0)
    def _():
        sum_ref[...] = jnp.zeros((128, H), jnp.float32)
        cnt_ref[...] = jnp.zeros((128, 128), jnp.float32)

    oh = oh_ref[...]
    sum_ref[...] += lax.dot_general(oh, h_ref[...], (((0,), (0,)), ((), ())),
                                    preferred_element_type=jnp.float32)
    cnt_ref[...] += lax.dot_general(oh, jnp.ones((BN, 128), jnp.float32),
                                    (((0,), (0,)), ((), ())),
                                    preferred_element_type=jnp.float32)


def _pool(h3, batch_oh):
    return pl.pallas_call(
        _pool_body,
        grid=(GRID,),
        in_specs=[pl.BlockSpec((BN, H), lambda i: (i, 0)),
                  pl.BlockSpec((BN, 128), lambda i: (i, 0))],
        out_specs=[pl.BlockSpec((128, H), lambda i: (0, 0)),
                   pl.BlockSpec((128, 128), lambda i: (0, 0))],
        out_shape=[jax.ShapeDtypeStruct((128, H), jnp.float32),
                   jax.ShapeDtypeStruct((128, 128), jnp.float32)],
    )(h3, batch_oh)


def _head_body(sum_ref, cnt_ref, w1, b1, w2, b2, w3, b3, w4, b4, w5, b5,
               o_ref):
    hg = sum_ref[...] / jnp.maximum(cnt_ref[:, 0:1], 1.0)
    x = jnp.maximum(jnp.dot(hg, w1[...], preferred_element_type=jnp.float32)
                    + b1[...], 0.0)
    x = jnp.maximum(jnp.dot(x, w2[...], preferred_element_type=jnp.float32)
                    + b2[...], 0.0)
    x = jnp.maximum(jnp.dot(x, w3[...], preferred_element_type=jnp.float32)
                    + b3[...], 0.0)
    x = jnp.maximum(jnp.dot(x, w4[...], preferred_element_type=jnp.float32)
                    + b4[...], 0.0)
    o_ref[...] = jnp.dot(x, w5[...], preferred_element_type=jnp.float32) \
        + b5[...]


def _head(sums, cnts, ws_and_bs):
    return pl.pallas_call(
        _head_body,
        out_shape=jax.ShapeDtypeStruct((128, 128), jnp.float32),
    )(sums, cnts, *ws_and_bs)


def kernel(x_node, edge_attr, edge_index, batch,
           W1, b1, W2, b2, W3, b3,
           g1, be1, g2, be2, g3, be3,
           L1w, L1b, L2w, L2b, L3w, L3b, L4w, L4b, L5w, L5b):
    del edge_attr
    f32 = jnp.float32

    ept0 = E // TILES
    npad = EPT - ept0
    padv = (N + (jnp.arange(npad, dtype=jnp.int32) % 8))[None, :]
    padv = jnp.broadcast_to(padv, (TILES, npad))
    srcp = jnp.concatenate(
        [edge_index[0].reshape(TILES, ept0), padv], axis=1
    ).reshape(TILES, NWIN, WIN)
    dstp = jnp.concatenate(
        [edge_index[1].reshape(TILES, ept0), padv], axis=1
    ).reshape(TILES, NWIN, WIN)

    x_pad = jnp.pad(x_node, ((0, NP - N), (0, 0)))
    batch_pad = jnp.pad(batch, (0, NP - N), constant_values=G)
    batch_oh = (batch_pad[:, None]
                == jnp.arange(128, dtype=batch.dtype)[None, :]).astype(f32)

    b1r, b2r, b3r = b1[None, :], b2[None, :], b3[None, :]
    g1r, g2r, g3r = g1[None, :], g2[None, :], g3[None, :]
    be1r, be2r, be3r = be1[None, :], be2[None, :], be3[None, :]
    head_args = (L1w, L1b[None, :], L2w, L2b[None, :], L3w, L3b[None, :],
                 L4w, L4b[None, :],
                 jnp.pad(L5w, ((0, 0), (0, 128 - 3))),
                 jnp.pad(L5b, (0, 128 - 3))[None, :])

    ones_tab = jnp.ones((NP, 128), f32)
    deg = _get_prop(0, 128, 128)(ones_tab, dstp)[:, :16]

    z0 = _scale_chunk(x_pad, deg)
    u0 = _get_prop(1, 128, 256)(z0, srcp, dstp)
    y1 = _mm1(u0, W1, b1r, deg)
    st1 = _stats(y1, deg, b1r, False)
    h1 = _apply(y1, deg, b1r, st1, g1r, be1r, False, True)

    z1 = _mm_chunk(h1, W2, deg)
    u2 = _get_prop(2, 128, 512)(z1, srcp, dstp)
    st2 = _stats(u2, deg, b2r, True)
    h2 = _apply(u2, deg, b2r, st2, g2r, be2r, True, True)

    z2 = _mm_chunk(h2, W3, deg)
    u3 = _get_prop(2, 128, 512)(z2, srcp, dstp)
    st3 = _stats(u3, deg, b3r, True)
    h3 = _apply(u3, deg, b3r, st3, g3r, be3r, True, False)

    sums, cnts = _pool(h3, batch_oh)
    out = _head(sums, cnts, head_args)
    return out[:G, :3]


# speedup vs baseline: 1.1421x; 1.1421x over previous
import functools

import jax
import jax.numpy as jnp
from jax import lax
from jax.experimental import pallas as pl
from jax.experimental.pallas import tpu as pltpu
from jax.experimental.pallas import tpu_sc as plsc

N = 10000
E = 160000
F = 256
H = 512
G = 64

NP = 10240
TILES = 16
ROWS_PT = NP // TILES
EPT = 10240
WIN = 128
NWIN = EPT // WIN
HWIN = NWIN // 2
BN = 256
GRID = NP // BN


def _make_prop(cpc, wc, out_w):
    mesh = plsc.VectorSubcoreMesh(core_axis_name="c", subcore_axis_name="s",
                                  num_cores=2, num_subcores=16)
    dense_out = wc % 128 == 0
    out_shape = (NP, out_w) if dense_out else (2 * cpc, NP, wc)

    @functools.partial(
        pl.kernel,
        out_type=jax.ShapeDtypeStruct(out_shape, jnp.float32),
        mesh=mesh,
        scratch_types=[
            pltpu.VMEM((HWIN, WIN), jnp.int32),
            pltpu.VMEM((HWIN, WIN), jnp.int32),
            pltpu.VMEM((2, WIN, wc), jnp.float32),
            pltpu.VMEM_SHARED((NP, wc), jnp.float32),
            pltpu.SemaphoreType.DMA,
            pltpu.SemaphoreType.DMA,
            pltpu.SemaphoreType.DMA,
            pltpu.SemaphoreType.DMA,
        ],
    )
    def prop(table, srcp, dstp, out, src_v, dst_v, gbuf, acc,
             sem0, sem1, sem2, sem3):
        c = lax.axis_index("c")
        s = lax.axis_index("s")
        r0 = s * ROWS_PT
        sems = (sem0, sem1)
        ssems = (sem2, sem3)
        for k in range(cpc):
            ch = c * cpc + k
            pltpu.sync_copy(table.at[ch, pl.ds(r0, ROWS_PT), :],
                            acc.at[pl.ds(r0, ROWS_PT), :])
            plsc.subcore_barrier()

            def start_g(j, b):
                pltpu.async_copy(table.at[ch].at[src_v.at[j]],
                                 gbuf.at[b], sems[b])

            def wait_g(b):
                pltpu.make_async_copy(table.at[ch].at[src_v.at[0]],
                                      gbuf.at[b], sems[b]).wait()

            def start_s(j, b):
                pltpu.async_copy(gbuf.at[b], acc.at[dst_v.at[j]],
                                 ssems[b], add=True)

            def wait_s(b):
                pltpu.make_async_copy(gbuf.at[b], acc.at[dst_v.at[0]],
                                      ssems[b]).wait()

            for h in range(2):
                pltpu.sync_copy(srcp.at[s, pl.ds(h * HWIN, HWIN)], src_v)
                pltpu.sync_copy(dstp.at[s, pl.ds(h * HWIN, HWIN)], dst_v)
                start_g(0, 0)
                start_g(1, 1)

                def win_body(w2, carry):
                    w = 2 * w2
                    for b in range(2):
                        wait_g(b)
                        pltpu.sync_copy(gbuf.at[b], acc.at[dst_v.at[w + b]],
                                        add=True)
                        start_g(jnp.minimum(w + 2 + b, HWIN - 1), b)
                    return carry

                lax.fori_loop(0, HWIN // 2, win_body, 0)
                wait_g(0)
                wait_g(1)
            plsc.subcore_barrier()
            if dense_out:
                pltpu.sync_copy(acc.at[pl.ds(r0, ROWS_PT), :],
                                out.at[pl.ds(r0, ROWS_PT), pl.ds(ch * wc, wc)])
            else:
                pltpu.sync_copy(acc.at[pl.ds(r0, ROWS_PT), :],
                                out.at[ch, pl.ds(r0, ROWS_PT), :])
            plsc.subcore_barrier()

    return prop


def _make_deg():
    mesh = plsc.VectorSubcoreMesh(core_axis_name="c", subcore_axis_name="s",
                                  num_cores=2, num_subcores=16)

    @functools.partial(
        pl.kernel,
        out_type=jax.ShapeDtypeStruct((NP, 128), jnp.float32),
        mesh=mesh,
        scratch_types=[
            pltpu.VMEM((NWIN, WIN), jnp.int32),
            pltpu.VMEM((WIN, 128), jnp.float32),
            pltpu.VMEM_SHARED((NP, 128), jnp.float32),
            pltpu.SemaphoreType.DMA,
            pltpu.SemaphoreType.DMA,
        ],
    )
    def degk(ones_tab, dstp, out, dst_v, gbuf, acc, sem0, sem1):
        c = lax.axis_index("c")
        s = lax.axis_index("s")
        pltpu.sync_copy(dstp.at[s], dst_v)
        pltpu.sync_copy(ones_tab.at[pl.ds(0, WIN), :], gbuf)
        r0 = s * ROWS_PT
        pltpu.sync_copy(ones_tab.at[pl.ds(r0, ROWS_PT), :],
                        acc.at[pl.ds(r0, ROWS_PT), :])
        plsc.subcore_barrier()
        ssems = (sem0, sem1)

        def start_s(j, b):
            pltpu.async_copy(gbuf, acc.at[dst_v.at[j]], ssems[b], add=True)

        def wait_s(b):
            pltpu.make_async_copy(gbuf, acc.at[dst_v.at[0]], ssems[b]).wait()

        start_s(0, 0)
        start_s(1, 1)

        def win_body(w2, carry):
            w = 2 * w2
            for b in range(2):
                wait_s(b)
                start_s(w + 2 + b, b)
            return carry

        lax.fori_loop(0, NWIN // 2 - 1, win_body, 0)
        wait_s(0)
        wait_s(1)
        plsc.subcore_barrier()

        @pl.when(c == 0)
        def _():
            pltpu.sync_copy(acc.at[pl.ds(r0, ROWS_PT), :],
                            out.at[pl.ds(r0, ROWS_PT), :])

    return degk


_PROP_CACHE = {}


def _get_prop(cpc, wc, out_w):
    key = (cpc, wc, out_w)
    if key not in _PROP_CACHE:
        _PROP_CACHE[key] = _make_prop(cpc, wc, out_w) if cpc else _make_deg()
    return _PROP_CACHE[key]


def _scale_chunk_body(x_ref, deg_ref, o_ref):
    dinv = lax.rsqrt(deg_ref[:, 0:1])
    x = x_ref[...]
    o_ref[0] = x[:, :128] * dinv
    o_ref[1] = x[:, 128:] * dinv


def _scale_chunk(x_pad, deg):
    return pl.pallas_call(
        _scale_chunk_body,
        grid=(GRID,),
        in_specs=[pl.BlockSpec((BN, F), lambda i: (i, 0)),
                  pl.BlockSpec((BN, 16), lambda i: (i, 0))],
        out_specs=pl.BlockSpec((2, BN, 128), lambda i: (0, i, 0)),
        out_shape=jax.ShapeDtypeStruct((2, NP, 128), jnp.float32),
    )(x_pad, deg)


def _mm1_body(u_ref, w_ref, b_ref, deg_ref, o_ref):
    dinv = lax.rsqrt(deg_ref[:, 0:1])
    y = jnp.dot(u_ref[...] * dinv, w_ref[...],
                preferred_element_type=jnp.float32)
    o_ref[...] = y + b_ref[...]


def _mm1(u0, w1, b1, deg):
    return pl.pallas_call(
        _mm1_body,
        grid=(GRID,),
        in_specs=[pl.BlockSpec((BN, F), lambda i: (i, 0)),
                  pl.BlockSpec((F, H), lambda i: (0, 0)),
                  pl.BlockSpec((1, H), lambda i: (0, 0)),
                  pl.BlockSpec((BN, 16), lambda i: (i, 0))],
        out_specs=pl.BlockSpec((BN, H), lambda i: (i, 0)),
        out_shape=jax.ShapeDtypeStruct((NP, H), jnp.float32),
    )(u0, w1, b1, deg)


def _mm_chunk_body(h_ref, w_ref, deg_ref, o_ref):
    z = jnp.dot(h_ref[...], w_ref[...], preferred_element_type=jnp.float32)
    z = z * lax.rsqrt(deg_ref[:, 0:1])
    for j in range(4):
        o_ref[j] = z[:, j * 128:(j + 1) * 128]


def _mm_chunk(h, w, deg):
    return pl.pallas_call(
        _mm_chunk_body,
        grid=(GRID,),
        in_specs=[pl.BlockSpec((BN, H), lambda i: (i, 0)),
                  pl.BlockSpec((H, H), lambda i: (0, 0)),
                  pl.BlockSpec((BN, 16), lambda i: (i, 0))],
        out_specs=pl.BlockSpec((4, BN, 128), lambda i: (0, i, 0)),
        out_shape=jax.ShapeDtypeStruct((4, NP, 128), jnp.float32),
    )(h, w, deg)


def _stats_body(y_ref, deg_ref, b_ref, o_ref, *, scale):
    i = pl.program_id(0)
    y = y_ref[...]
    if scale:
        y = y * lax.rsqrt(deg_ref[:, 0:1]) + b_ref[...]
    rows = lax.broadcasted_iota(jnp.int32, (BN, 1), 0) + i * BN
    y = jnp.where(rows < N, y, 0.0)

    @pl.when(i == 0)
    def _():
        o_ref[...] = jnp.zeros((8, H), jnp.float32)

    s1 = jnp.sum(y, axis=0)[None, :]
    s2 = jnp.sum(y * y, axis=0)[None, :]
    o_ref[0:2, :] += jnp.concatenate([s1, s2], axis=0)


def _stats(y, deg, b, scale):
    return pl.pallas_call(
        functools.partial(_stats_body, scale=scale),
        grid=(GRID,),
        in_specs=[pl.BlockSpec((BN, H), lambda i: (i, 0)),
                  pl.BlockSpec((BN, 16), lambda i: (i, 0)),
                  pl.BlockSpec((1, H), lambda i: (0, 0))],
        out_specs=pl.BlockSpec((8, H), lambda i: (0, 0)),
        out_shape=jax.ShapeDtypeStruct((8, H), jnp.float32),
    )(y, deg, b)


def _apply_body(y_ref, deg_ref, b_ref, st_ref, g_ref, be_ref, o_ref,
                *, scale, relu):
    i = pl.program_id(0)
    y = y_ref[...]
    if scale:
        y = y * lax.rsqrt(deg_ref[:, 0:1]) + b_ref[...]
    mu = st_ref[0:1, :] / N
    var = st_ref[1:2, :] / N - mu * mu
    out = g_ref[...] * (y - mu) * lax.rsqrt(var + 1e-5) + be_ref[...]
    if relu:
        out = jnp.maximum(out, 0.0)
    rows = lax.broadcasted_iota(jnp.int32, (BN, 1), 0) + i * BN
    o_ref[...] = jnp.where(rows < N, out, 0.0)


def _apply(y, deg, b, st, g, be, scale, relu):
    return pl.pallas_call(
        functools.partial(_apply_body, scale=scale, relu=relu),
        grid=(GRID,),
        in_specs=[pl.BlockSpec((BN, H), lambda i: (i, 0)),
                  pl.BlockSpec((BN, 16), lambda i: (i, 0)),
                  pl.BlockSpec((1, H), lambda i: (0, 0)),
                  pl.BlockSpec((8, H), lambda i: (0, 0)),
                  pl.BlockSpec((1, H), lambda i: (0, 0)),
                  pl.BlockSpec((1, H), lambda i: (0, 0))],
        out_specs=pl.BlockSpec((BN, H), lambda i: (i, 0)),
        out_shape=jax.ShapeDtypeStruct((NP, H), jnp.float32),
    )(y, deg, b, st, g, be)


def _pool_body(h_ref, oh_ref, sum_ref, cnt_ref):
    i = pl.program_id(0)

    @pl.when(i == 0)
    def _():
        sum_ref[...] = jnp.zeros((128, H), jnp.float32)
        cnt_ref[...] = jnp.zeros((128, 128), jnp.float32)

    oh = oh_ref[...]
    sum_ref[...] += lax.dot_general(oh, h_ref[...], (((0,), (0,)), ((), ())),
                                    preferred_element_type=jnp.float32)
    cnt_ref[...] += lax.dot_general(oh, jnp.ones((BN, 128), jnp.float32),
                                    (((0,), (0,)), ((), ())),
                                    preferred_element_type=jnp.float32)


def _pool(h3, batch_oh):
    return pl.pallas_call(
        _pool_body,
        grid=(GRID,),
        in_specs=[pl.BlockSpec((BN, H), lambda i: (i, 0)),
                  pl.BlockSpec((BN, 128), lambda i: (i, 0))],
        out_specs=[pl.BlockSpec((128, H), lambda i: (0, 0)),
                   pl.BlockSpec((128, 128), lambda i: (0, 0))],
        out_shape=[jax.ShapeDtypeStruct((128, H), jnp.float32),
                   jax.ShapeDtypeStruct((128, 128), jnp.float32)],
    )(h3, batch_oh)


def _head_body(sum_ref, cnt_ref, w1, b1, w2, b2, w3, b3, w4, b4, w5, b5,
               o_ref):
    hg = sum_ref[...] / jnp.maximum(cnt_ref[:, 0:1], 1.0)
    x = jnp.maximum(jnp.dot(hg, w1[...], preferred_element_type=jnp.float32)
                    + b1[...], 0.0)
    x = jnp.maximum(jnp.dot(x, w2[...], preferred_element_type=jnp.float32)
                    + b2[...], 0.0)
    x = jnp.maximum(jnp.dot(x, w3[...], preferred_element_type=jnp.float32)
                    + b3[...], 0.0)
    x = jnp.maximum(jnp.dot(x, w4[...], preferred_element_type=jnp.float32)
                    + b4[...], 0.0)
    o_ref[...] = jnp.dot(x, w5[...], preferred_element_type=jnp.float32) \
        + b5[...]


def _head(sums, cnts, ws_and_bs):
    return pl.pallas_call(
        _head_body,
        out_shape=jax.ShapeDtypeStruct((128, 128), jnp.float32),
    )(sums, cnts, *ws_and_bs)


def kernel(x_node, edge_attr, edge_index, batch,
           W1, b1, W2, b2, W3, b3,
           g1, be1, g2, be2, g3, be3,
           L1w, L1b, L2w, L2b, L3w, L3b, L4w, L4b, L5w, L5b):
    del edge_attr
    f32 = jnp.float32

    ept0 = E // TILES
    npad = EPT - ept0
    padv = (N + (jnp.arange(npad, dtype=jnp.int32) % 8))[None, :]
    padv = jnp.broadcast_to(padv, (TILES, npad))
    srcp = jnp.concatenate(
        [edge_index[0].reshape(TILES, ept0), padv], axis=1
    ).reshape(TILES, NWIN, WIN)
    dstp = jnp.concatenate(
        [edge_index[1].reshape(TILES, ept0), padv], axis=1
    ).reshape(TILES, NWIN, WIN)

    x_pad = jnp.pad(x_node, ((0, NP - N), (0, 0)))
    batch_pad = jnp.pad(batch, (0, NP - N), constant_values=G)
    batch_oh = (batch_pad[:, None]
                == jnp.arange(128, dtype=batch.dtype)[None, :]).astype(f32)

    b1r, b2r, b3r = b1[None, :], b2[None, :], b3[None, :]
    g1r, g2r, g3r = g1[None, :], g2[None, :], g3[None, :]
    be1r, be2r, be3r = be1[None, :], be2[None, :], be3[None, :]
    head_args = (L1w, L1b[None, :], L2w, L2b[None, :], L3w, L3b[None, :],
                 L4w, L4b[None, :],
                 jnp.pad(L5w, ((0, 0), (0, 128 - 3))),
                 jnp.pad(L5b, (0, 128 - 3))[None, :])

    ones_tab = jnp.ones((NP, 128), f32)
    deg = _get_prop(0, 128, 128)(ones_tab, dstp)[:, :16]

    z0 = _scale_chunk(x_pad, deg)
    u0 = _get_prop(1, 128, 256)(z0, srcp, dstp)
    y1 = _mm1(u0, W1, b1r, deg)
    st1 = _stats(y1, deg, b1r, False)
    h1 = _apply(y1, deg, b1r, st1, g1r, be1r, False, True)

    z1 = _mm_chunk(h1, W2, deg)
    u2 = _get_prop(2, 128, 512)(z1, srcp, dstp)
    st2 = _stats(u2, deg, b2r, True)
    h2 = _apply(u2, deg, b2r, st2, g2r, be2r, True, True)

    z2 = _mm_chunk(h2, W3, deg)
    u3 = _get_prop(2, 128, 512)(z2, srcp, dstp)
    st3 = _stats(u3, deg, b3r, True)
    h3 = _apply(u3, deg, b3r, st3, g3r, be3r, True, False)

    sums, cnts = _pool(h3, batch_oh)
    out = _head(sums, cnts, head_args)
    return out[:G, :3]
